# baseline (speedup 1.0000x reference)
import functools

import jax
import jax.numpy as jnp
from jax import lax
from jax.experimental import pallas as pl
from jax.experimental.pallas import tpu as pltpu
from jax.experimental.pallas import tpu_sc as plsc

_CHUNKS = (32, 112, 112, 112, 112, 32)


def kernel(t, table):
    (B,) = t.shape
    V, D = table.shape

    info = plsc.get_sparse_core_info()
    NC, NS = info.num_cores, info.num_subcores
    NW = NC * NS

    per_w = B // NW
    assert per_w * NW == B and sum(_CHUNKS) == per_w
    offs = [sum(_CHUNKS[:j]) for j in range(len(_CHUNKS))]
    assert all(o % 8 == 0 for o in offs) and all(c <= 128 for c in _CHUNKS)

    rpt = ((V + NS - 1) // NS + 7) // 8 * 8
    n_full = V // rpt
    rem = V - n_full * rpt
    assert rem % 8 == 0 and V % 8 == 0

    idx = t.reshape(NW, per_w)
    mesh = plsc.VectorSubcoreMesh(core_axis_name="c", subcore_axis_name="s")

    @functools.partial(
        pl.kernel,
        mesh=mesh,
        out_type=jax.ShapeDtypeStruct((B, D), jnp.float32),
        scratch_types=[
            pltpu.VMEM((per_w,), jnp.int32),
            pltpu.VMEM((per_w, D), jnp.float32),
            pltpu.VMEM_SHARED((V, D), jnp.float32),
            pltpu.SemaphoreType.DMA,
            pltpu.SemaphoreType.DMA,
        ],
    )
    def emb(table_hbm, idx_hbm, out_hbm, idx_v, rows_v, table_sp, gsem, wsem):
        cid = lax.axis_index("c")
        sid = lax.axis_index("s")
        wid = sid * NC + cid
        idx_cp = pltpu.async_copy(idx_hbm.at[wid], idx_v, wsem)

        @pl.when(sid < n_full)
        def _():
            pltpu.sync_copy(
                table_hbm.at[pl.ds(sid * rpt, rpt)],
                table_sp.at[pl.ds(sid * rpt, rpt)],
            )

        if rem:

            @pl.when(sid == n_full)
            def _():
                pltpu.sync_copy(
                    table_hbm.at[pl.ds(n_full * rpt, rem)],
                    table_sp.at[pl.ds(n_full * rpt, rem)],
                )

        plsc.subcore_barrier()
        idx_cp.wait()
        gathers = [
            pltpu.async_copy(
                table_sp.at[idx_v.at[pl.ds(o, c)]],
                rows_v.at[pl.ds(o, c)],
                gsem,
            )
            for o, c in zip(offs, _CHUNKS)
        ]
        writes = []
        for g, o, c in zip(gathers, offs, _CHUNKS):
            g.wait()
            writes.append(
                pltpu.async_copy(
                    rows_v.at[pl.ds(o, c)],
                    out_hbm.at[pl.ds(wid * per_w + o, c)],
                    wsem,
                )
            )
        for w in writes:
            w.wait()

    return emb(table, idx)


# speedup vs baseline: 1.0108x; 1.0108x over previous
import functools

import jax
import jax.numpy as jnp
from jax import lax
from jax.experimental import pallas as pl
from jax.experimental.pallas import tpu as pltpu
from jax.experimental.pallas import tpu_sc as plsc

_CHUNK = 128


def kernel(t, table):
    (B,) = t.shape
    V, D = table.shape

    info = plsc.get_sparse_core_info()
    NC, NS = info.num_cores, info.num_subcores
    NW = NC * NS

    n_chunks = B // (NW * _CHUNK)
    assert B == NW * n_chunks * _CHUNK

    rpt = ((V + NS - 1) // NS + 7) // 8 * 8
    n_full = V // rpt
    rem = V - n_full * rpt
    assert rem % 8 == 0 and V % 8 == 0

    idx = t.reshape(NW, n_chunks, _CHUNK)
    mesh = plsc.VectorSubcoreMesh(core_axis_name="c", subcore_axis_name="s")

    @functools.partial(
        pl.kernel,
        mesh=mesh,
        out_type=jax.ShapeDtypeStruct((NW, n_chunks, _CHUNK, D), jnp.float32),
        scratch_types=[
            pltpu.VMEM((n_chunks, _CHUNK), jnp.int32),
            pltpu.VMEM((n_chunks, _CHUNK, D), jnp.float32),
            pltpu.VMEM_SHARED((V, D), jnp.float32),
            pltpu.SemaphoreType.DMA,
            pltpu.SemaphoreType.DMA,
        ],
    )
    def emb(table_hbm, idx_hbm, out_hbm, idx_v, rows_v, table_sp, gsem, wsem):
        cid = lax.axis_index("c")
        sid = lax.axis_index("s")
        wid = sid * NC + cid
        idx_cp = pltpu.async_copy(idx_hbm.at[wid], idx_v, wsem)

        @pl.when(sid < n_full)
        def _():
            pltpu.sync_copy(
                table_hbm.at[pl.ds(sid * rpt, rpt)],
                table_sp.at[pl.ds(sid * rpt, rpt)],
            )

        if rem:

            @pl.when(sid == n_full)
            def _():
                pltpu.sync_copy(
                    table_hbm.at[pl.ds(n_full * rpt, rem)],
                    table_sp.at[pl.ds(n_full * rpt, rem)],
                )

        idx_cp.wait()
        plsc.subcore_barrier()
        gathers = [
            pltpu.async_copy(table_sp.at[idx_v.at[j]], rows_v.at[j], gsem)
            for j in range(n_chunks)
        ]
        writes = []
        for j in range(n_chunks):
            gathers[j].wait()
            writes.append(pltpu.async_copy(rows_v.at[j], out_hbm.at[wid, j], wsem))
        for w in writes:
            w.wait()

    return emb(table, idx).reshape(B, D)
